# Initial kernel scaffold; baseline (speedup 1.0000x reference)
import jax
import jax.numpy as jnp
from jax.experimental import pallas as pl


def kernel(h_V, h_E, edge_idx, W1w, W1b, W2w, W2b, W3w, W3b, g1, b1, d1w, d1b, d2w, d2b, g2, b2):
    raise NotImplementedError("write your pallas kernel here")


# speedup vs baseline: 1.7628x; 1.7628x over previous
import functools

import jax
import jax.numpy as jnp
from jax import lax
from jax.experimental import pallas as pl
from jax.experimental.pallas import tpu as pltpu
from jax.experimental.pallas import tpu_sc as plsc

N = 10000
E = 320000
H = 128
HIN = 144


BE = 2000


def _edge_mlp_body(he_ref, w1_ref, b1_ref, w2_ref, b2_ref, w3_ref, b3_ref, out_ref):
    x = he_ref[...]
    h1 = jnp.maximum(jnp.dot(x, w1_ref[...], preferred_element_type=jnp.float32) + b1_ref[...], 0.0)
    h2 = jnp.maximum(jnp.dot(h1, w2_ref[...], preferred_element_type=jnp.float32) + b2_ref[...], 0.0)
    y = jnp.dot(h2, w3_ref[...], preferred_element_type=jnp.float32) + b3_ref[...]
    out_ref[...] = y * (1.0 / 30.0)


def _edge_mlp(h_E, W1w, W1b, W2w, W2b, W3w, W3b):
    return pl.pallas_call(
        _edge_mlp_body,
        grid=(E // BE,),
        in_specs=[
            pl.BlockSpec((BE, HIN), lambda i: (i, 0)),
            pl.BlockSpec((HIN, H), lambda i: (0, 0)),
            pl.BlockSpec((1, H), lambda i: (0, 0)),
            pl.BlockSpec((H, H), lambda i: (0, 0)),
            pl.BlockSpec((1, H), lambda i: (0, 0)),
            pl.BlockSpec((H, H), lambda i: (0, 0)),
            pl.BlockSpec((1, H), lambda i: (0, 0)),
        ],
        out_specs=pl.BlockSpec((BE, H), lambda i: (i, 0)),
        out_shape=jax.ShapeDtypeStruct((E, H), jnp.float32),
    )(h_E, W1w, W1b, W2w, W2b, W3w, W3b)


NC = 2
NS = 16
NW = NC * NS
PER_TILE = E // NW
CH = 128
NFULL = PER_TILE // CH
REM = PER_TILE - NFULL * CH
CZ = 80
NCHUNKS = N // CZ


def _scatter_partials(m, src_idx):
    mesh = plsc.VectorSubcoreMesh(core_axis_name="c", subcore_axis_name="s")

    @functools.partial(
        pl.kernel,
        mesh=mesh,
        out_type=jax.ShapeDtypeStruct((NC, N, H), jnp.float32),
        scratch_types=[
            pltpu.VMEM((CH,), jnp.int32),
            pltpu.VMEM((CH, H), jnp.float32),
            pltpu.VMEM((REM,), jnp.int32),
            pltpu.VMEM((REM, H), jnp.float32),
            pltpu.VMEM((CZ, H), jnp.float32),
            pltpu.VMEM_SHARED((N, H), jnp.float32),
        ],
    )
    def sc_kernel(m_hbm, idx_hbm, zero_hbm, out_hbm, ibuf, mbuf, irem, mrem, zbuf, acc):
        c = lax.axis_index("c")
        s = lax.axis_index("s")
        nk = lax.select(s < NCHUNKS % NS, NCHUNKS // NS + 1, NCHUNKS // NS)
        pltpu.sync_copy(zero_hbm, zbuf)

        def zero_body(k, carry):
            pltpu.sync_copy(zbuf, acc.at[pl.ds((s + k * NS) * CZ, CZ)])
            return carry

        lax.fori_loop(0, nk, zero_body, 0)
        plsc.subcore_barrier()
        base = (c * NS + s) * PER_TILE

        def body(i, carry):
            off = base + i * CH
            pltpu.sync_copy(idx_hbm.at[pl.ds(off, CH)], ibuf)
            pltpu.sync_copy(m_hbm.at[pl.ds(off, CH)], mbuf)
            pltpu.sync_copy(mbuf, acc.at[ibuf], add=True)
            return carry

        lax.fori_loop(0, NFULL, body, 0)
        off = base + NFULL * CH
        pltpu.sync_copy(idx_hbm.at[pl.ds(off, REM)], irem)
        pltpu.sync_copy(m_hbm.at[pl.ds(off, REM)], mrem)
        pltpu.sync_copy(mrem, acc.at[irem], add=True)
        plsc.subcore_barrier()
        def out_body(k, carry):
            sl = pl.ds((s + k * NS) * CZ, CZ)
            pltpu.sync_copy(acc.at[sl], zbuf)
            pltpu.sync_copy(zbuf, out_hbm.at[c, sl])
            return carry

        lax.fori_loop(0, nk, out_body, 0)

    zero = jnp.zeros((CZ, H), jnp.float32)
    return sc_kernel(m, src_idx, zero)


BN = 2000


def _node_body(hv_ref, p0_ref, p1_ref, d1w_ref, d1b_ref, d2w_ref, d2b_ref,
               g1_ref, b1_ref, g2_ref, b2_ref, out_ref):
    x = hv_ref[...] + p0_ref[...] + p1_ref[...]
    mu = jnp.mean(x, axis=-1, keepdims=True)
    xc = x - mu
    var = jnp.mean(xc * xc, axis=-1, keepdims=True)
    hv1 = xc * lax.rsqrt(var + 1e-5) * g1_ref[...] + b1_ref[...]
    t = jnp.maximum(jnp.dot(hv1, d1w_ref[...], preferred_element_type=jnp.float32) + d1b_ref[...], 0.0)
    x2 = hv1 + jnp.dot(t, d2w_ref[...], preferred_element_type=jnp.float32) + d2b_ref[...]
    mu2 = jnp.mean(x2, axis=-1, keepdims=True)
    xc2 = x2 - mu2
    var2 = jnp.mean(xc2 * xc2, axis=-1, keepdims=True)
    out_ref[...] = xc2 * lax.rsqrt(var2 + 1e-5) * g2_ref[...] + b2_ref[...]


def _node_update(h_V, p0, p1, d1w, d1b, d2w, d2b, g1, b1, g2, b2):
    full = lambda shape: pl.BlockSpec(shape, lambda i: tuple(0 for _ in shape))
    return pl.pallas_call(
        _node_body,
        grid=(N // BN,),
        in_specs=[
            pl.BlockSpec((BN, H), lambda i: (i, 0)),
            pl.BlockSpec((BN, H), lambda i: (i, 0)),
            pl.BlockSpec((BN, H), lambda i: (i, 0)),
            full((H, 4 * H)),
            full((1, 4 * H)),
            full((4 * H, H)),
            full((1, H)),
            full((1, H)),
            full((1, H)),
            full((1, H)),
            full((1, H)),
        ],
        out_specs=pl.BlockSpec((BN, H), lambda i: (i, 0)),
        out_shape=jax.ShapeDtypeStruct((N, H), jnp.float32),
    )(h_V, p0, p1, d1w, d1b, d2w, d2b, g1, b1, g2, b2)


def kernel(h_V, h_E, edge_idx, W1w, W1b, W2w, W2b, W3w, W3b, g1, b1, d1w, d1b, d2w, d2b, g2, b2):
    row = lambda v: v.reshape(1, -1)
    m = _edge_mlp(h_E, W1w, row(W1b), W2w, row(W2b), W3w, row(W3b))
    partials = _scatter_partials(m, edge_idx[0])
    return _node_update(h_V, partials[0], partials[1], d1w, row(d1b), d2w, row(d2b),
                        row(g1), row(b1), row(g2), row(b2))


# speedup vs baseline: 1.9846x; 1.1258x over previous
import functools

import jax
import jax.numpy as jnp
from jax import lax
from jax.experimental import pallas as pl
from jax.experimental.pallas import tpu as pltpu
from jax.experimental.pallas import tpu_sc as plsc

N = 10000
E = 320000
H = 128
HIN = 144


BE = 2000


def _edge_mlp_body(he_ref, w1_ref, b1_ref, w2_ref, b2_ref, w3_ref, b3_ref, out_ref):
    x = he_ref[...]
    h1 = jnp.maximum(jnp.dot(x, w1_ref[...], preferred_element_type=jnp.float32) + b1_ref[...], 0.0)
    h2 = jnp.maximum(jnp.dot(h1, w2_ref[...], preferred_element_type=jnp.float32) + b2_ref[...], 0.0)
    y = jnp.dot(h2, w3_ref[...], preferred_element_type=jnp.float32) + b3_ref[...]
    out_ref[...] = y * (1.0 / 30.0)


def _edge_mlp(h_E, W1w, W1b, W2w, W2b, W3w, W3b):
    return pl.pallas_call(
        _edge_mlp_body,
        grid=(E // BE,),
        in_specs=[
            pl.BlockSpec((BE, HIN), lambda i: (i, 0)),
            pl.BlockSpec((HIN, H), lambda i: (0, 0)),
            pl.BlockSpec((1, H), lambda i: (0, 0)),
            pl.BlockSpec((H, H), lambda i: (0, 0)),
            pl.BlockSpec((1, H), lambda i: (0, 0)),
            pl.BlockSpec((H, H), lambda i: (0, 0)),
            pl.BlockSpec((1, H), lambda i: (0, 0)),
        ],
        out_specs=pl.BlockSpec((BE, H), lambda i: (i, 0)),
        out_shape=jax.ShapeDtypeStruct((E, H), jnp.float32),
    )(h_E, W1w, W1b, W2w, W2b, W3w, W3b)


NC = 2
NS = 16
NW = NC * NS
PER_TILE = E // NW
CH = 128
NFULL = PER_TILE // CH
REM = PER_TILE - NFULL * CH
CZ = 80
NCHUNKS = N // CZ


def _scatter_partials(m, src_idx):
    mesh = plsc.VectorSubcoreMesh(core_axis_name="c", subcore_axis_name="s")

    @functools.partial(
        pl.kernel,
        mesh=mesh,
        out_type=jax.ShapeDtypeStruct((NC, N, H), jnp.float32),
        scratch_types=[
            pltpu.VMEM((CH,), jnp.int32),
            pltpu.VMEM((CH, H), jnp.float32),
            pltpu.VMEM((CH,), jnp.int32),
            pltpu.VMEM((CH, H), jnp.float32),
            pltpu.VMEM((REM,), jnp.int32),
            pltpu.VMEM((REM, H), jnp.float32),
            pltpu.VMEM((CZ, H), jnp.float32),
            pltpu.VMEM_SHARED((N, H), jnp.float32),
            pltpu.SemaphoreType.DMA,
            pltpu.SemaphoreType.DMA,
        ],
    )
    def sc_kernel(m_hbm, idx_hbm, zero_hbm, out_hbm,
                  ibuf_a, mbuf_a, ibuf_b, mbuf_b, irem, mrem, zbuf, acc,
                  sem_a, sem_b):
        c = lax.axis_index("c")
        s = lax.axis_index("s")
        nk = lax.select(s < NCHUNKS % NS, NCHUNKS // NS + 1, NCHUNKS // NS)
        pltpu.sync_copy(zero_hbm, zbuf)

        def zero_body(k, carry):
            pltpu.sync_copy(zbuf, acc.at[pl.ds((s + k * NS) * CZ, CZ)])
            return carry

        lax.fori_loop(0, nk, zero_body, 0)
        plsc.subcore_barrier()
        base = (c * NS + s) * PER_TILE

        def load(ch, ib, mb, sem):
            off = base + ch * CH
            pltpu.async_copy(idx_hbm.at[pl.ds(off, CH)], ib, sem)
            pltpu.async_copy(m_hbm.at[pl.ds(off, CH)], mb, sem)

        def wait(ib, mb, sem):
            pltpu.make_async_copy(idx_hbm.at[pl.ds(0, CH)], ib, sem).wait()
            pltpu.make_async_copy(m_hbm.at[pl.ds(0, CH)], mb, sem).wait()

        load(0, ibuf_a, mbuf_a, sem_a)

        def body(j, carry):
            k = 2 * j
            wait(ibuf_a, mbuf_a, sem_a)
            load(k + 1, ibuf_b, mbuf_b, sem_b)
            pltpu.sync_copy(mbuf_a, acc.at[ibuf_a], add=True)
            wait(ibuf_b, mbuf_b, sem_b)

            @pl.when(k + 2 < NFULL)
            def _():
                load(k + 2, ibuf_a, mbuf_a, sem_a)

            pltpu.sync_copy(mbuf_b, acc.at[ibuf_b], add=True)
            return carry

        lax.fori_loop(0, NFULL // 2, body, 0)
        off = base + NFULL * CH
        pltpu.sync_copy(idx_hbm.at[pl.ds(off, REM)], irem)
        pltpu.sync_copy(m_hbm.at[pl.ds(off, REM)], mrem)
        pltpu.sync_copy(mrem, acc.at[irem], add=True)
        plsc.subcore_barrier()
        def out_body(k, carry):
            sl = pl.ds((s + k * NS) * CZ, CZ)
            pltpu.sync_copy(acc.at[sl], zbuf)
            pltpu.sync_copy(zbuf, out_hbm.at[c, sl])
            return carry

        lax.fori_loop(0, nk, out_body, 0)

    zero = jnp.zeros((CZ, H), jnp.float32)
    return sc_kernel(m, src_idx, zero)


BN = 2000


def _node_body(hv_ref, p0_ref, p1_ref, d1w_ref, d1b_ref, d2w_ref, d2b_ref,
               g1_ref, b1_ref, g2_ref, b2_ref, out_ref):
    x = hv_ref[...] + p0_ref[...] + p1_ref[...]
    mu = jnp.mean(x, axis=-1, keepdims=True)
    xc = x - mu
    var = jnp.mean(xc * xc, axis=-1, keepdims=True)
    hv1 = xc * lax.rsqrt(var + 1e-5) * g1_ref[...] + b1_ref[...]
    t = jnp.maximum(jnp.dot(hv1, d1w_ref[...], preferred_element_type=jnp.float32) + d1b_ref[...], 0.0)
    x2 = hv1 + jnp.dot(t, d2w_ref[...], preferred_element_type=jnp.float32) + d2b_ref[...]
    mu2 = jnp.mean(x2, axis=-1, keepdims=True)
    xc2 = x2 - mu2
    var2 = jnp.mean(xc2 * xc2, axis=-1, keepdims=True)
    out_ref[...] = xc2 * lax.rsqrt(var2 + 1e-5) * g2_ref[...] + b2_ref[...]


def _node_update(h_V, p0, p1, d1w, d1b, d2w, d2b, g1, b1, g2, b2):
    full = lambda shape: pl.BlockSpec(shape, lambda i: tuple(0 for _ in shape))
    return pl.pallas_call(
        _node_body,
        grid=(N // BN,),
        in_specs=[
            pl.BlockSpec((BN, H), lambda i: (i, 0)),
            pl.BlockSpec((BN, H), lambda i: (i, 0)),
            pl.BlockSpec((BN, H), lambda i: (i, 0)),
            full((H, 4 * H)),
            full((1, 4 * H)),
            full((4 * H, H)),
            full((1, H)),
            full((1, H)),
            full((1, H)),
            full((1, H)),
            full((1, H)),
        ],
        out_specs=pl.BlockSpec((BN, H), lambda i: (i, 0)),
        out_shape=jax.ShapeDtypeStruct((N, H), jnp.float32),
    )(h_V, p0, p1, d1w, d1b, d2w, d2b, g1, b1, g2, b2)


def kernel(h_V, h_E, edge_idx, W1w, W1b, W2w, W2b, W3w, W3b, g1, b1, d1w, d1b, d2w, d2b, g2, b2):
    row = lambda v: v.reshape(1, -1)
    m = _edge_mlp(h_E, W1w, row(W1b), W2w, row(W2b), W3w, row(W3b))
    partials = _scatter_partials(m, edge_idx[0])
    return _node_update(h_V, partials[0], partials[1], d1w, row(d1b), d2w, row(d2b),
                        row(g1), row(b1), row(g2), row(b2))


# speedup vs baseline: 2.1383x; 1.0775x over previous
import functools

import jax
import jax.numpy as jnp
from jax import lax
from jax.experimental import pallas as pl
from jax.experimental.pallas import tpu as pltpu
from jax.experimental.pallas import tpu_sc as plsc

N = 10000
E = 320000
H = 128
HIN = 144

KS = 2
ES = E // KS


BE = 2000


def _edge_mlp_body(he_ref, w1_ref, b1_ref, w2_ref, b2_ref, w3_ref, b3_ref, out_ref):
    x = he_ref[...]
    h1 = jnp.maximum(jnp.dot(x, w1_ref[...], preferred_element_type=jnp.float32) + b1_ref[...], 0.0)
    h2 = jnp.maximum(jnp.dot(h1, w2_ref[...], preferred_element_type=jnp.float32) + b2_ref[...], 0.0)
    y = jnp.dot(h2, w3_ref[...], preferred_element_type=jnp.float32) + b3_ref[...]
    out_ref[...] = y * (1.0 / 30.0)


def _edge_mlp(h_E, W1w, W1b, W2w, W2b, W3w, W3b, grp):
    blk0 = grp * (ES // BE)
    return pl.pallas_call(
        _edge_mlp_body,
        grid=(ES // BE,),
        in_specs=[
            pl.BlockSpec((BE, HIN), lambda i: (blk0 + i, 0)),
            pl.BlockSpec((HIN, H), lambda i: (0, 0)),
            pl.BlockSpec((1, H), lambda i: (0, 0)),
            pl.BlockSpec((H, H), lambda i: (0, 0)),
            pl.BlockSpec((1, H), lambda i: (0, 0)),
            pl.BlockSpec((H, H), lambda i: (0, 0)),
            pl.BlockSpec((1, H), lambda i: (0, 0)),
        ],
        out_specs=pl.BlockSpec((BE, H), lambda i: (i, 0)),
        out_shape=jax.ShapeDtypeStruct((ES, H), jnp.float32),
    )(h_E, W1w, W1b, W2w, W2b, W3w, W3b)


NC = 2
NS = 16
NW = NC * NS
PER_TILE = ES // NW
CH = 128
NFULL = PER_TILE // CH
REM = PER_TILE - NFULL * CH
CZ = 80
NCHUNKS = N // CZ


def _scatter_partials(m, src_idx, zero, grp):
    mesh = plsc.VectorSubcoreMesh(core_axis_name="c", subcore_axis_name="s")
    grp_off = grp * ES

    @functools.partial(
        pl.kernel,
        mesh=mesh,
        out_type=jax.ShapeDtypeStruct((NC, N, H), jnp.float32),
        scratch_types=[
            pltpu.VMEM((CH,), jnp.int32),
            pltpu.VMEM((CH, H), jnp.float32),
            pltpu.VMEM((CH,), jnp.int32),
            pltpu.VMEM((CH, H), jnp.float32),
            pltpu.VMEM((REM,), jnp.int32),
            pltpu.VMEM((REM, H), jnp.float32),
            pltpu.VMEM((CZ, H), jnp.float32),
            pltpu.VMEM_SHARED((N, H), jnp.float32),
            pltpu.SemaphoreType.DMA,
            pltpu.SemaphoreType.DMA,
        ],
    )
    def sc_kernel(m_hbm, idx_hbm, zero_hbm, out_hbm,
                  ibuf_a, mbuf_a, ibuf_b, mbuf_b, irem, mrem, zbuf, acc,
                  sem_a, sem_b):
        c = lax.axis_index("c")
        s = lax.axis_index("s")
        nk = lax.select(s < NCHUNKS % NS, NCHUNKS // NS + 1, NCHUNKS // NS)
        pltpu.sync_copy(zero_hbm, zbuf)

        def zero_body(k, carry):
            pltpu.sync_copy(zbuf, acc.at[pl.ds((s + k * NS) * CZ, CZ)])
            return carry

        lax.fori_loop(0, nk, zero_body, 0)
        plsc.subcore_barrier()
        base = (c * NS + s) * PER_TILE

        def load(ch, ib, mb, sem):
            off = base + ch * CH
            pltpu.async_copy(idx_hbm.at[pl.ds(grp_off + off, CH)], ib, sem)
            pltpu.async_copy(m_hbm.at[pl.ds(off, CH)], mb, sem)

        def wait(ib, mb, sem):
            pltpu.make_async_copy(idx_hbm.at[pl.ds(0, CH)], ib, sem).wait()
            pltpu.make_async_copy(m_hbm.at[pl.ds(0, CH)], mb, sem).wait()

        load(0, ibuf_a, mbuf_a, sem_a)

        def body(j, carry):
            k = 2 * j
            wait(ibuf_a, mbuf_a, sem_a)
            load(k + 1, ibuf_b, mbuf_b, sem_b)
            pltpu.sync_copy(mbuf_a, acc.at[ibuf_a], add=True)
            wait(ibuf_b, mbuf_b, sem_b)

            @pl.when(k + 2 < NFULL)
            def _():
                load(k + 2, ibuf_a, mbuf_a, sem_a)

            pltpu.sync_copy(mbuf_b, acc.at[ibuf_b], add=True)
            return carry

        lax.fori_loop(0, NFULL // 2, body, 0)
        if NFULL % 2 == 1:
            wait(ibuf_a, mbuf_a, sem_a)
            pltpu.sync_copy(mbuf_a, acc.at[ibuf_a], add=True)
        if REM:
            off = base + NFULL * CH
            pltpu.sync_copy(idx_hbm.at[pl.ds(grp_off + off, REM)], irem)
            pltpu.sync_copy(m_hbm.at[pl.ds(off, REM)], mrem)
            pltpu.sync_copy(mrem, acc.at[irem], add=True)
        plsc.subcore_barrier()
        def out_body(k, carry):
            sl = pl.ds((s + k * NS) * CZ, CZ)
            pltpu.sync_copy(acc.at[sl], zbuf)
            pltpu.sync_copy(zbuf, out_hbm.at[c, sl])
            return carry

        lax.fori_loop(0, nk, out_body, 0)

    return sc_kernel(m, src_idx, zero)


BN = 2000


def _node_body(*refs):
    hv_ref = refs[0]
    parts = refs[1:1 + 2 * KS]
    d1w_ref, d1b_ref, d2w_ref, d2b_ref, g1_ref, b1_ref, g2_ref, b2_ref = refs[1 + 2 * KS:-1]
    out_ref = refs[-1]
    x = hv_ref[...]
    for p in parts:
        x = x + p[0]
    mu = jnp.mean(x, axis=-1, keepdims=True)
    xc = x - mu
    var = jnp.mean(xc * xc, axis=-1, keepdims=True)
    hv1 = xc * lax.rsqrt(var + 1e-5) * g1_ref[...] + b1_ref[...]
    t = jnp.maximum(jnp.dot(hv1, d1w_ref[...], preferred_element_type=jnp.float32) + d1b_ref[...], 0.0)
    x2 = hv1 + jnp.dot(t, d2w_ref[...], preferred_element_type=jnp.float32) + d2b_ref[...]
    mu2 = jnp.mean(x2, axis=-1, keepdims=True)
    xc2 = x2 - mu2
    var2 = jnp.mean(xc2 * xc2, axis=-1, keepdims=True)
    out_ref[...] = xc2 * lax.rsqrt(var2 + 1e-5) * g2_ref[...] + b2_ref[...]


def _node_update(h_V, partials, d1w, d1b, d2w, d2b, g1, b1, g2, b2):
    full = lambda shape: pl.BlockSpec(shape, lambda i: tuple(0 for _ in shape))
    node_blk = pl.BlockSpec((BN, H), lambda i: (i, 0))
    flat_parts = []
    part_specs = []
    for p in partials:
        for cidx in range(NC):
            flat_parts.append(p)
            part_specs.append(pl.BlockSpec((1, BN, H), lambda i, c=cidx: (c, i, 0)))
    return pl.pallas_call(
        _node_body,
        grid=(N // BN,),
        in_specs=[node_blk] + part_specs + [
            full((H, 4 * H)),
            full((1, 4 * H)),
            full((4 * H, H)),
            full((1, H)),
            full((1, H)),
            full((1, H)),
            full((1, H)),
            full((1, H)),
        ],
        out_specs=node_blk,
        out_shape=jax.ShapeDtypeStruct((N, H), jnp.float32),
    )(h_V, *flat_parts, d1w, d1b, d2w, d2b, g1, b1, g2, b2)


def kernel(h_V, h_E, edge_idx, W1w, W1b, W2w, W2b, W3w, W3b, g1, b1, d1w, d1b, d2w, d2b, g2, b2):
    row = lambda v: v.reshape(1, -1)
    src = edge_idx[0]
    zero = jnp.zeros((CZ, H), jnp.float32)
    partials = []
    for k in range(KS):
        m_k = _edge_mlp(h_E, W1w, row(W1b), W2w, row(W2b), W3w, row(W3b), k)
        partials.append(_scatter_partials(m_k, src, zero, k))
    return _node_update(h_V, partials, d1w, row(d1b), d2w, row(d2b),
                        row(g1), row(b1), row(g2), row(b2))


# speedup vs baseline: 4.0652x; 1.9011x over previous
import functools

import jax
import jax.numpy as jnp
from jax import lax
from jax.experimental import pallas as pl
from jax.experimental.pallas import tpu as pltpu
from jax.experimental.pallas import tpu_sc as plsc

N = 10000
E = 320000
H = 128
HIN = 144

KS = 2
ES = E // KS


BE = 3200


def _edge_mlp_body(he_ref, w1_ref, b1_ref, w2_ref, b2_ref, w3_ref, b3_ref, out_ref):
    xt = he_ref[...]
    h1 = jnp.maximum(
        lax.dot_general(xt, w1_ref[...], (((0,), (0,)), ((), ())),
                        preferred_element_type=jnp.float32) + b1_ref[...], 0.0)
    h2 = jnp.maximum(jnp.dot(h1, w2_ref[...], preferred_element_type=jnp.float32) + b2_ref[...], 0.0)
    y = jnp.dot(h2, w3_ref[...], preferred_element_type=jnp.float32) + b3_ref[...]
    out_ref[...] = y * (1.0 / 30.0)


def _edge_mlp(h_ET, W1w, W1b, W2w, W2b, W3w, W3b, grp):
    blk0 = grp * (ES // BE)
    return pl.pallas_call(
        _edge_mlp_body,
        grid=(ES // BE,),
        in_specs=[
            pl.BlockSpec((HIN, BE), lambda i: (0, blk0 + i)),
            pl.BlockSpec((HIN, H), lambda i: (0, 0)),
            pl.BlockSpec((1, H), lambda i: (0, 0)),
            pl.BlockSpec((H, H), lambda i: (0, 0)),
            pl.BlockSpec((1, H), lambda i: (0, 0)),
            pl.BlockSpec((H, H), lambda i: (0, 0)),
            pl.BlockSpec((1, H), lambda i: (0, 0)),
        ],
        out_specs=pl.BlockSpec((BE, H), lambda i: (i, 0)),
        out_shape=jax.ShapeDtypeStruct((ES, H), jnp.float32),
    )(h_ET, W1w, W1b, W2w, W2b, W3w, W3b)


NC = 2
NS = 16
NW = NC * NS
PER_TILE = ES // NW
CH = 128
NFULL = PER_TILE // CH
REM = PER_TILE - NFULL * CH
CZ = 80
NCHUNKS = N // CZ


def _scatter_partials(m, src_idx, zero, grp):
    mesh = plsc.VectorSubcoreMesh(core_axis_name="c", subcore_axis_name="s")
    grp_off = grp * ES

    @functools.partial(
        pl.kernel,
        mesh=mesh,
        out_type=jax.ShapeDtypeStruct((NC, N, H), jnp.float32),
        scratch_types=[
            pltpu.VMEM((CH,), jnp.int32),
            pltpu.VMEM((CH, H), jnp.float32),
            pltpu.VMEM((CH,), jnp.int32),
            pltpu.VMEM((CH, H), jnp.float32),
            pltpu.VMEM((REM,), jnp.int32),
            pltpu.VMEM((REM, H), jnp.float32),
            pltpu.VMEM((CZ, H), jnp.float32),
            pltpu.VMEM_SHARED((N, H), jnp.float32),
            pltpu.SemaphoreType.DMA,
            pltpu.SemaphoreType.DMA,
        ],
    )
    def sc_kernel(m_hbm, idx_hbm, zero_hbm, out_hbm,
                  ibuf_a, mbuf_a, ibuf_b, mbuf_b, irem, mrem, zbuf, acc,
                  sem_a, sem_b):
        c = lax.axis_index("c")
        s = lax.axis_index("s")
        nk = lax.select(s < NCHUNKS % NS, NCHUNKS // NS + 1, NCHUNKS // NS)
        pltpu.sync_copy(zero_hbm, zbuf)

        def zero_body(k, carry):
            pltpu.sync_copy(zbuf, acc.at[pl.ds((s + k * NS) * CZ, CZ)])
            return carry

        lax.fori_loop(0, nk, zero_body, 0)
        plsc.subcore_barrier()
        base = (c * NS + s) * PER_TILE

        def load(ch, ib, mb, sem):
            off = base + ch * CH
            pltpu.async_copy(idx_hbm.at[pl.ds(grp_off + off, CH)], ib, sem)
            pltpu.async_copy(m_hbm.at[pl.ds(off, CH)], mb, sem)

        def wait(ib, mb, sem):
            pltpu.make_async_copy(idx_hbm.at[pl.ds(0, CH)], ib, sem).wait()
            pltpu.make_async_copy(m_hbm.at[pl.ds(0, CH)], mb, sem).wait()

        load(0, ibuf_a, mbuf_a, sem_a)

        def body(j, carry):
            k = 2 * j
            wait(ibuf_a, mbuf_a, sem_a)
            load(k + 1, ibuf_b, mbuf_b, sem_b)
            pltpu.sync_copy(mbuf_a, acc.at[ibuf_a], add=True)
            wait(ibuf_b, mbuf_b, sem_b)

            @pl.when(k + 2 < NFULL)
            def _():
                load(k + 2, ibuf_a, mbuf_a, sem_a)

            pltpu.sync_copy(mbuf_b, acc.at[ibuf_b], add=True)
            return carry

        lax.fori_loop(0, NFULL // 2, body, 0)
        if NFULL % 2 == 1:
            wait(ibuf_a, mbuf_a, sem_a)
            pltpu.sync_copy(mbuf_a, acc.at[ibuf_a], add=True)
        if REM:
            off = base + NFULL * CH
            pltpu.sync_copy(idx_hbm.at[pl.ds(grp_off + off, REM)], irem)
            pltpu.sync_copy(m_hbm.at[pl.ds(off, REM)], mrem)
            pltpu.sync_copy(mrem, acc.at[irem], add=True)
        plsc.subcore_barrier()
        def out_body(k, carry):
            sl = pl.ds((s + k * NS) * CZ, CZ)
            pltpu.sync_copy(acc.at[sl], zbuf)
            pltpu.sync_copy(zbuf, out_hbm.at[c, sl])
            return carry

        lax.fori_loop(0, nk, out_body, 0)

    return sc_kernel(m, src_idx, zero)


BN = 2000


def _node_body(*refs):
    hv_ref = refs[0]
    parts = refs[1:1 + 2 * KS]
    d1w_ref, d1b_ref, d2w_ref, d2b_ref, g1_ref, b1_ref, g2_ref, b2_ref = refs[1 + 2 * KS:-1]
    out_ref = refs[-1]
    x = hv_ref[...]
    for p in parts:
        x = x + p[0]
    mu = jnp.mean(x, axis=-1, keepdims=True)
    xc = x - mu
    var = jnp.mean(xc * xc, axis=-1, keepdims=True)
    hv1 = xc * lax.rsqrt(var + 1e-5) * g1_ref[...] + b1_ref[...]
    t = jnp.maximum(jnp.dot(hv1, d1w_ref[...], preferred_element_type=jnp.float32) + d1b_ref[...], 0.0)
    x2 = hv1 + jnp.dot(t, d2w_ref[...], preferred_element_type=jnp.float32) + d2b_ref[...]
    mu2 = jnp.mean(x2, axis=-1, keepdims=True)
    xc2 = x2 - mu2
    var2 = jnp.mean(xc2 * xc2, axis=-1, keepdims=True)
    out_ref[...] = xc2 * lax.rsqrt(var2 + 1e-5) * g2_ref[...] + b2_ref[...]


def _node_update(h_V, partials, d1w, d1b, d2w, d2b, g1, b1, g2, b2):
    full = lambda shape: pl.BlockSpec(shape, lambda i: tuple(0 for _ in shape))
    node_blk = pl.BlockSpec((BN, H), lambda i: (i, 0))
    flat_parts = []
    part_specs = []
    for p in partials:
        for cidx in range(NC):
            flat_parts.append(p)
            part_specs.append(pl.BlockSpec((1, BN, H), lambda i, c=cidx: (c, i, 0)))
    return pl.pallas_call(
        _node_body,
        grid=(N // BN,),
        in_specs=[node_blk] + part_specs + [
            full((H, 4 * H)),
            full((1, 4 * H)),
            full((4 * H, H)),
            full((1, H)),
            full((1, H)),
            full((1, H)),
            full((1, H)),
            full((1, H)),
        ],
        out_specs=node_blk,
        out_shape=jax.ShapeDtypeStruct((N, H), jnp.float32),
    )(h_V, *flat_parts, d1w, d1b, d2w, d2b, g1, b1, g2, b2)


def kernel(h_V, h_E, edge_idx, W1w, W1b, W2w, W2b, W3w, W3b, g1, b1, d1w, d1b, d2w, d2b, g2, b2):
    row = lambda v: v.reshape(1, -1)
    src = edge_idx[0]
    zero = jnp.zeros((CZ, H), jnp.float32)
    h_ET = h_E.T
    partials = []
    for k in range(KS):
        m_k = _edge_mlp(h_ET, W1w, row(W1b), W2w, row(W2b), W3w, row(W3b), k)
        partials.append(_scatter_partials(m_k, src, zero, k))
    return _node_update(h_V, partials, d1w, row(d1b), d2w, row(d2b),
                        row(g1), row(b1), row(g2), row(b2))


# speedup vs baseline: 4.1752x; 1.0271x over previous
import functools

import jax
import jax.numpy as jnp
from jax import lax
from jax.experimental import pallas as pl
from jax.experimental.pallas import tpu as pltpu
from jax.experimental.pallas import tpu_sc as plsc

N = 10000
E = 320000
H = 128
HIN = 144

GROUPS = (128000, 128000, 64000)
STARTS = (0, 128000, 256000)


BE = 6400


def _edge_mlp_body(he_ref, w1_ref, b1_ref, w2_ref, b2_ref, w3_ref, b3_ref, out_ref):
    xt = he_ref[...]
    h1 = jnp.maximum(
        lax.dot_general(xt, w1_ref[...], (((0,), (0,)), ((), ())),
                        preferred_element_type=jnp.float32) + b1_ref[...], 0.0)
    h2 = jnp.maximum(jnp.dot(h1, w2_ref[...], preferred_element_type=jnp.float32) + b2_ref[...], 0.0)
    y = jnp.dot(h2, w3_ref[...], preferred_element_type=jnp.float32) + b3_ref[...]
    out_ref[...] = y * (1.0 / 30.0)


def _edge_mlp(h_ET, W1w, W1b, W2w, W2b, W3w, W3b, start, size):
    blk0 = start // BE
    return pl.pallas_call(
        _edge_mlp_body,
        grid=(size // BE,),
        in_specs=[
            pl.BlockSpec((HIN, BE), lambda i: (0, blk0 + i)),
            pl.BlockSpec((HIN, H), lambda i: (0, 0)),
            pl.BlockSpec((1, H), lambda i: (0, 0)),
            pl.BlockSpec((H, H), lambda i: (0, 0)),
            pl.BlockSpec((1, H), lambda i: (0, 0)),
            pl.BlockSpec((H, H), lambda i: (0, 0)),
            pl.BlockSpec((1, H), lambda i: (0, 0)),
        ],
        out_specs=pl.BlockSpec((BE, H), lambda i: (i, 0)),
        out_shape=jax.ShapeDtypeStruct((size, H), jnp.float32),
    )(h_ET, W1w, W1b, W2w, W2b, W3w, W3b)


NC = 2
NS = 16
NW = NC * NS
CH = 128
CZ = 80
NCHUNKS = N // CZ


def _scatter_partials(m, src_idx, zero, start, size):
    mesh = plsc.VectorSubcoreMesh(core_axis_name="c", subcore_axis_name="s")
    per_tile = size // NW
    nfull = per_tile // CH
    rem = per_tile - nfull * CH

    @functools.partial(
        pl.kernel,
        mesh=mesh,
        out_type=jax.ShapeDtypeStruct((NC, N, H), jnp.float32),
        scratch_types=[
            pltpu.VMEM((CH,), jnp.int32),
            pltpu.VMEM((CH, H), jnp.float32),
            pltpu.VMEM((CH,), jnp.int32),
            pltpu.VMEM((CH, H), jnp.float32),
            pltpu.VMEM((max(rem, 8),), jnp.int32),
            pltpu.VMEM((CZ, H), jnp.float32),
            pltpu.VMEM_SHARED((N, H), jnp.float32),
            pltpu.SemaphoreType.DMA,
            pltpu.SemaphoreType.DMA,
        ],
    )
    def sc_kernel(m_hbm, idx_hbm, zero_hbm, out_hbm,
                  ibuf_a, mbuf_a, ibuf_b, mbuf_b, irem, zbuf, acc,
                  sem_a, sem_b):
        c = lax.axis_index("c")
        s = lax.axis_index("s")
        nk = lax.select(s < NCHUNKS % NS, NCHUNKS // NS + 1, NCHUNKS // NS)
        pltpu.sync_copy(zero_hbm, zbuf)

        def zero_body(k, carry):
            pltpu.sync_copy(zbuf, acc.at[pl.ds((s + k * NS) * CZ, CZ)])
            return carry

        lax.fori_loop(0, nk, zero_body, 0)
        plsc.subcore_barrier()
        base = (c * NS + s) * per_tile

        def load(ch, ib, mb, sem):
            off = base + ch * CH
            pltpu.async_copy(idx_hbm.at[pl.ds(start + off, CH)], ib, sem)
            pltpu.async_copy(m_hbm.at[pl.ds(off, CH)], mb, sem)

        def wait(ib, mb, sem):
            pltpu.make_async_copy(idx_hbm.at[pl.ds(0, CH)], ib, sem).wait()
            pltpu.make_async_copy(m_hbm.at[pl.ds(0, CH)], mb, sem).wait()

        load(0, ibuf_a, mbuf_a, sem_a)

        def body(j, carry):
            k = 2 * j
            wait(ibuf_a, mbuf_a, sem_a)
            load(k + 1, ibuf_b, mbuf_b, sem_b)
            pltpu.sync_copy(mbuf_a, acc.at[ibuf_a], add=True)
            wait(ibuf_b, mbuf_b, sem_b)

            @pl.when(k + 2 < nfull)
            def _():
                load(k + 2, ibuf_a, mbuf_a, sem_a)

            pltpu.sync_copy(mbuf_b, acc.at[ibuf_b], add=True)
            return carry

        lax.fori_loop(0, nfull // 2, body, 0)
        if nfull % 2 == 1:
            wait(ibuf_a, mbuf_a, sem_a)
            pltpu.sync_copy(mbuf_a, acc.at[ibuf_a], add=True)
        if rem:
            off = base + nfull * CH
            pltpu.sync_copy(idx_hbm.at[pl.ds(start + off, rem)], irem)
            pltpu.sync_copy(m_hbm.at[pl.ds(off, rem)], mbuf_a.at[pl.ds(0, rem)])
            pltpu.sync_copy(mbuf_a.at[pl.ds(0, rem)], acc.at[irem], add=True)
        plsc.subcore_barrier()
        def out_body(k, carry):
            sl = pl.ds((s + k * NS) * CZ, CZ)
            pltpu.sync_copy(acc.at[sl], zbuf)
            pltpu.sync_copy(zbuf, out_hbm.at[c, sl])
            return carry

        lax.fori_loop(0, nk, out_body, 0)

    return sc_kernel(m, src_idx, zero)


BN = 2000
NPART = len(GROUPS) * NC


def _node_body(*refs):
    hv_ref = refs[0]
    parts = refs[1:1 + NPART]
    d1w_ref, d1b_ref, d2w_ref, d2b_ref, g1_ref, b1_ref, g2_ref, b2_ref = refs[1 + NPART:-1]
    out_ref = refs[-1]
    x = hv_ref[...]
    for p in parts:
        x = x + p[0]
    mu = jnp.mean(x, axis=-1, keepdims=True)
    xc = x - mu
    var = jnp.mean(xc * xc, axis=-1, keepdims=True)
    hv1 = xc * lax.rsqrt(var + 1e-5) * g1_ref[...] + b1_ref[...]
    t = jnp.maximum(jnp.dot(hv1, d1w_ref[...], preferred_element_type=jnp.float32) + d1b_ref[...], 0.0)
    x2 = hv1 + jnp.dot(t, d2w_ref[...], preferred_element_type=jnp.float32) + d2b_ref[...]
    mu2 = jnp.mean(x2, axis=-1, keepdims=True)
    xc2 = x2 - mu2
    var2 = jnp.mean(xc2 * xc2, axis=-1, keepdims=True)
    out_ref[...] = xc2 * lax.rsqrt(var2 + 1e-5) * g2_ref[...] + b2_ref[...]


def _node_update(h_V, partials, d1w, d1b, d2w, d2b, g1, b1, g2, b2):
    full = lambda shape: pl.BlockSpec(shape, lambda i: tuple(0 for _ in shape))
    node_blk = pl.BlockSpec((BN, H), lambda i: (i, 0))
    flat_parts = []
    part_specs = []
    for p in partials:
        for cidx in range(NC):
            flat_parts.append(p)
            part_specs.append(pl.BlockSpec((1, BN, H), lambda i, c=cidx: (c, i, 0)))
    return pl.pallas_call(
        _node_body,
        grid=(N // BN,),
        in_specs=[node_blk] + part_specs + [
            full((H, 4 * H)),
            full((1, 4 * H)),
            full((4 * H, H)),
            full((1, H)),
            full((1, H)),
            full((1, H)),
            full((1, H)),
            full((1, H)),
        ],
        out_specs=node_blk,
        out_shape=jax.ShapeDtypeStruct((N, H), jnp.float32),
    )(h_V, *flat_parts, d1w, d1b, d2w, d2b, g1, b1, g2, b2)


def kernel(h_V, h_E, edge_idx, W1w, W1b, W2w, W2b, W3w, W3b, g1, b1, d1w, d1b, d2w, d2b, g2, b2):
    row = lambda v: v.reshape(1, -1)
    src = edge_idx[0]
    zero = jnp.zeros((CZ, H), jnp.float32)
    h_ET = h_E.T
    partials = []
    for start, size in zip(STARTS, GROUPS):
        m_k = _edge_mlp(h_ET, W1w, row(W1b), W2w, row(W2b), W3w, row(W3b), start, size)
        partials.append(_scatter_partials(m_k, src, zero, start, size))
    return _node_update(h_V, partials, d1w, row(d1b), d2w, row(d2b),
                        row(g1), row(b1), row(g2), row(b2))
